# baseline (speedup 1.0000x reference)
import functools

import jax
import jax.numpy as jnp
from jax.experimental import pallas as pl
from jax.experimental.pallas import tpu as pltpu

VB = 1024
MIN_FREQ_THRESHOLD = 0.01


def _tc_body(targets_ref, tt_ref, logits_ref, boost_ref, td_ref, tok_ref,
             out_ref):
    j = pl.program_id(0)
    base = j * VB
    tgt = targets_ref[...].reshape(256, 1)
    vocab_ids = base + jax.lax.broadcasted_iota(jnp.int32, (1, VB), 1)
    eq = (tgt == vocab_ids).astype(jnp.float32)
    counts = jnp.sum(eq, axis=0)

    tc = tok_ref[...] + counts
    tt = jnp.maximum(tt_ref[0] + 256.0, 1.0)
    cur = tc / tt
    ratio = cur / jnp.maximum(td_ref[...], 1e-8)
    boost = boost_ref[...]
    boost_new = jnp.where(ratio < MIN_FREQ_THRESHOLD, boost * 1.1,
                          boost * 0.99)
    logb = jnp.log(boost_new)
    out_ref[...] = logits_ref[...] + logb[None, :]


@jax.jit
def kernel(logits, targets, common_word_boost, target_dist, token_counts,
           total_tokens):
    B, S, V = logits.shape
    n = B * S
    logits2 = logits.reshape(n, V)
    tgt = targets.reshape(n).astype(jnp.int32)
    nv = pl.cdiv(V, VB)

    vec_spec = pl.BlockSpec((VB,), lambda j: (j,))
    out2 = pl.pallas_call(
        _tc_body,
        grid=(nv,),
        in_specs=[
            pl.BlockSpec((n,), lambda j: (0,)),
            pl.BlockSpec(memory_space=pltpu.SMEM),
            pl.BlockSpec((n, VB), lambda j: (0, j)),
            vec_spec,
            vec_spec,
            vec_spec,
        ],
        out_specs=pl.BlockSpec((n, VB), lambda j: (0, j)),
        out_shape=jax.ShapeDtypeStruct((n, V), jnp.float32),
    )(tgt, total_tokens, logits2, common_word_boost, target_dist,
      token_counts)
    return out2.reshape(B, S, V)
